# baseline (speedup 1.0000x reference)
import functools

import jax
import jax.numpy as jnp
import numpy as np
from jax import lax
from jax.experimental import pallas as pl
from jax.experimental.pallas import tpu as pltpu
from jax.experimental.pallas import tpu_sc as plsc

N_NODES = 10000
D_FEAT = 128
DH = 64
DS = 80
N_PAD = 10016
NC, NS = 2, 16
CHUNK = 125
NCHUNK = 160
assert NS * NCHUNK * CHUNK == 320000
NG = 4
ROWS_PER_TILE = N_PAD // NS


def _sc_aggregate(x_bf, src4, dst3):
  mesh = plsc.VectorSubcoreMesh(
      core_axis_name="c", subcore_axis_name="s", num_cores=NC, num_subcores=NS)

  @functools.partial(
      pl.kernel,
      out_type=jax.ShapeDtypeStruct((NC, N_PAD, DS), jnp.float32),
      mesh=mesh,
      compiler_params=pltpu.CompilerParams(
          use_tc_tiling_on_sc=False, needs_layout_passes=False),
      scratch_types=[
          pltpu.VMEM((NCHUNK, CHUNK), jnp.int32),
          pltpu.VMEM((NCHUNK, CHUNK), jnp.int32),
          [pltpu.VMEM((CHUNK, DH), jnp.bfloat16)] * NG,
          [pltpu.VMEM((CHUNK, DS), jnp.float32)] * 2,
          pltpu.VMEM_SHARED((N_PAD, DS), jnp.float32),
          [pltpu.SemaphoreType.DMA] * NG,
          [pltpu.SemaphoreType.DMA] * 2,
      ],
  )
  def k(x_hbm, src_hbm, dst_hbm, out_hbm,
        src_v, dst_v, gbuf, sbuf, acc, gsem, ssem):
    c = lax.axis_index("c")
    s = lax.axis_index("s")

    def zero_row(r, _):
      for kk in range(DS // 16):
        sbuf[0][r, pl.ds(kk * 16, 16)] = jnp.zeros((16,), jnp.float32)
      return 0
    lax.fori_loop(0, CHUNK, zero_row, 0)
    base = s * ROWS_PER_TILE
    for i in range(ROWS_PER_TILE // CHUNK):
      pltpu.sync_copy(sbuf[0], acc.at[pl.ds(base + i * CHUNK, CHUNK)])
    rem = ROWS_PER_TILE % CHUNK
    if rem:
      pltpu.sync_copy(sbuf[0].at[pl.ds(0, rem)],
                      acc.at[pl.ds(base + ROWS_PER_TILE - rem, rem)])

    one0 = jnp.where(lax.iota(jnp.int32, 16) == 0, 1.0, 0.0).astype(jnp.float32)

    def ones_row(r, _):
      for sb in range(2):
        sbuf[sb][r, pl.ds(DH, 16)] = one0
      return 0
    lax.fori_loop(0, CHUNK, ones_row, 0)

    pltpu.sync_copy(src_hbm.at[c * NS + s], src_v)
    pltpu.sync_copy(dst_hbm.at[s], dst_v)
    plsc.subcore_barrier()

    for b in range(NG):
      pltpu.async_copy(x_hbm.at[src_v.at[b]], gbuf[b], gsem[b])

    def unpack_chunk(b, sb):
      def rows5(r5, _):
        for dr in range(5):
          r = r5 * 5 + dr
          for g in range(DH // 32):
            lo, hi = plsc.unpack(gbuf[b][r, pl.ds(32 * g, 32)],
                                 format=plsc.PackFormat.INTERLEAVED)
            sbuf[sb][r, pl.ds(32 * g, 16)] = lo
            sbuf[sb][r, pl.ds(32 * g + 16, 16)] = hi
        return 0
      lax.fori_loop(0, CHUNK // 5, rows5, 0)

    def body(grp, _):
      for b in range(NG):
        j = grp * NG + b
        sb = b % 2
        pltpu.make_async_copy(x_hbm.at[src_v.at[j]], gbuf[b], gsem[b]).wait()

        @pl.when(j >= 2)
        def _():
          pltpu.make_async_copy(
              sbuf[sb], acc.at[dst_v.at[j - 2]], ssem[sb]).wait()

        unpack_chunk(b, sb)
        pltpu.async_copy(sbuf[sb], acc.at[dst_v.at[j]], ssem[sb], add=True)

        @pl.when(j + NG < NCHUNK)
        def _():
          pltpu.async_copy(x_hbm.at[src_v.at[j + NG]], gbuf[b], gsem[b])

      return 0

    lax.fori_loop(0, NCHUNK // NG, body, 0)
    for sb in range(2):
      pltpu.make_async_copy(
          sbuf[sb], acc.at[dst_v.at[NCHUNK - 2 + sb]], ssem[sb]).wait()

    plsc.subcore_barrier()
    pltpu.sync_copy(acc.at[pl.ds(base, ROWS_PER_TILE)],
                    out_hbm.at[c, pl.ds(base, ROWS_PER_TILE)])

  return k(x_bf, src4, dst3)


def _tc_head(agg, W_mp, b_mp, W1, b1, W2, b2):
  BLK = 2000
  grid = N_NODES // BLK

  def body(a0_ref, a1_ref, wmp_ref, bmp_ref, w1_ref, b1_ref, w2_ref, b2_ref, out_ref):
    a0 = a0_ref[0]
    a1 = a1_ref[0]
    deg = jnp.maximum(jnp.sum(a0[:, DH:], axis=1, keepdims=True), 1.0)
    m = (jnp.dot(a0[:, :DH], wmp_ref[:DH, :], preferred_element_type=jnp.float32)
         + jnp.dot(a1[:, :DH], wmp_ref[DH:, :],
                   preferred_element_type=jnp.float32))
    h = jnp.maximum(m / deg + bmp_ref[...], 0.0)
    h = jnp.maximum(
        jnp.dot(h, w1_ref[...], preferred_element_type=jnp.float32)
        + b1_ref[...], 0.0)
    out_ref[...] = (
        jnp.dot(h, w2_ref[...], preferred_element_type=jnp.float32)
        + b2_ref[...])

  full = lambda shape: pl.BlockSpec(shape, lambda i: (0, 0))
  return pl.pallas_call(
      body,
      grid=(grid,),
      in_specs=[
          pl.BlockSpec((1, BLK, DS), lambda i: (0, i, 0)),
          pl.BlockSpec((1, BLK, DS), lambda i: (1, i, 0)),
          full((D_FEAT, D_FEAT)),
          full((1, D_FEAT)),
          full((D_FEAT, 256)),
          full((1, 256)),
          full((256, D_FEAT)),
          full((1, D_FEAT)),
      ],
      out_specs=pl.BlockSpec((BLK, D_FEAT), lambda i: (i, 0)),
      out_shape=jax.ShapeDtypeStruct((N_NODES, D_FEAT), jnp.float32),
  )(agg, agg, W_mp, b_mp, W1, b1, W2, b2)


@jax.jit
def kernel(features, edge_index, W_mp, b_mp, W1, b1, W2, b2):
  src = edge_index[0].astype(jnp.int32)
  dst = edge_index[1].astype(jnp.int32)
  src4 = jnp.concatenate([src, src + N_NODES]).reshape(NC * NS, NCHUNK, CHUNK)
  dst3 = dst.reshape(NS, NCHUNK, CHUNK)
  halves = features.reshape(N_NODES, 2, DH).transpose(1, 0, 2)
  x_bf = (halves.reshape(2 * N_NODES, DH // 32, 2, 16)
          .transpose(0, 1, 3, 2)
          .reshape(2 * N_NODES, DH)
          .astype(jnp.bfloat16))

  agg = _sc_aggregate(x_bf, src4, dst3)
  return _tc_head(agg, W_mp, b_mp.reshape(1, -1), W1, b1.reshape(1, -1),
                  W2, b2.reshape(1, -1))


# speedup vs baseline: 1.0042x; 1.0042x over previous
import functools

import jax
import jax.numpy as jnp
import numpy as np
from jax import lax
from jax.experimental import pallas as pl
from jax.experimental.pallas import tpu as pltpu
from jax.experimental.pallas import tpu_sc as plsc

N_NODES = 10000
D_FEAT = 128
DH = 64
DS = 80
N_PAD = 10016
NC, NS = 2, 16
CHUNK = 125
NCHUNK = 160
assert NS * NCHUNK * CHUNK == 320000
NG = 4
ROWS_PER_TILE = N_PAD // NS


def _sc_aggregate(x_bf, src4, dst3):
  mesh = plsc.VectorSubcoreMesh(
      core_axis_name="c", subcore_axis_name="s", num_cores=NC, num_subcores=NS)

  @functools.partial(
      pl.kernel,
      out_type=jax.ShapeDtypeStruct((NC, N_PAD, DS), jnp.float32),
      mesh=mesh,
      compiler_params=pltpu.CompilerParams(
          use_tc_tiling_on_sc=False, needs_layout_passes=False),
      scratch_types=[
          pltpu.VMEM((NCHUNK, CHUNK), jnp.int32),
          pltpu.VMEM((NCHUNK, CHUNK), jnp.int32),
          [pltpu.VMEM((CHUNK, DH), jnp.bfloat16)] * NG,
          [pltpu.VMEM((CHUNK, DS), jnp.float32)] * 2,
          pltpu.VMEM_SHARED((N_PAD, DS), jnp.float32),
          [pltpu.SemaphoreType.DMA] * NG,
          [pltpu.SemaphoreType.DMA] * 2,
      ],
  )
  def k(x_hbm, src_hbm, dst_hbm, out_hbm,
        src_v, dst_v, gbuf, sbuf, acc, gsem, ssem):
    c = lax.axis_index("c")
    s = lax.axis_index("s")

    pltpu.sync_copy(src_hbm.at[c * NS + s], src_v)
    pltpu.sync_copy(dst_hbm.at[s], dst_v)
    for b in range(NG):
      pltpu.async_copy(x_hbm.at[src_v.at[b]], gbuf[b], gsem[b])

    def zero_row(r, _):
      for kk in range(DS // 16):
        sbuf[0][r, pl.ds(kk * 16, 16)] = jnp.zeros((16,), jnp.float32)
      return 0
    lax.fori_loop(0, CHUNK, zero_row, 0)
    base = s * ROWS_PER_TILE
    for i in range(ROWS_PER_TILE // CHUNK):
      pltpu.sync_copy(sbuf[0], acc.at[pl.ds(base + i * CHUNK, CHUNK)])
    rem = ROWS_PER_TILE % CHUNK
    if rem:
      pltpu.sync_copy(sbuf[0].at[pl.ds(0, rem)],
                      acc.at[pl.ds(base + ROWS_PER_TILE - rem, rem)])

    one0 = jnp.where(lax.iota(jnp.int32, 16) == 0, 1.0, 0.0).astype(jnp.float32)

    def ones_row(r, _):
      for sb in range(2):
        sbuf[sb][r, pl.ds(DH, 16)] = one0
      return 0
    lax.fori_loop(0, CHUNK, ones_row, 0)

    plsc.subcore_barrier()


    def unpack_chunk(b, sb):
      def rows5(r5, _):
        for dr in range(5):
          r = r5 * 5 + dr
          for g in range(DH // 32):
            lo, hi = plsc.unpack(gbuf[b][r, pl.ds(32 * g, 32)],
                                 format=plsc.PackFormat.INTERLEAVED)
            sbuf[sb][r, pl.ds(32 * g, 16)] = lo
            sbuf[sb][r, pl.ds(32 * g + 16, 16)] = hi
        return 0
      lax.fori_loop(0, CHUNK // 5, rows5, 0)

    def body(grp, _):
      for b in range(NG):
        j = grp * NG + b
        sb = b % 2
        pltpu.make_async_copy(x_hbm.at[src_v.at[j]], gbuf[b], gsem[b]).wait()

        @pl.when(j >= 2)
        def _():
          pltpu.make_async_copy(
              sbuf[sb], acc.at[dst_v.at[j - 2]], ssem[sb]).wait()

        unpack_chunk(b, sb)
        pltpu.async_copy(sbuf[sb], acc.at[dst_v.at[j]], ssem[sb], add=True)

        @pl.when(j + NG < NCHUNK)
        def _():
          pltpu.async_copy(x_hbm.at[src_v.at[j + NG]], gbuf[b], gsem[b])

      return 0

    lax.fori_loop(0, NCHUNK // NG, body, 0)
    for sb in range(2):
      pltpu.make_async_copy(
          sbuf[sb], acc.at[dst_v.at[NCHUNK - 2 + sb]], ssem[sb]).wait()

    plsc.subcore_barrier()
    pltpu.sync_copy(acc.at[pl.ds(base, ROWS_PER_TILE)],
                    out_hbm.at[c, pl.ds(base, ROWS_PER_TILE)])

  return k(x_bf, src4, dst3)


def _tc_head(agg, W_mp, b_mp, W1, b1, W2, b2):
  BLK = 2000
  grid = N_NODES // BLK

  def body(a0_ref, a1_ref, wmp_ref, bmp_ref, w1_ref, b1_ref, w2_ref, b2_ref, out_ref):
    a0 = a0_ref[0]
    a1 = a1_ref[0]
    deg = jnp.maximum(jnp.sum(a0[:, DH:], axis=1, keepdims=True), 1.0)
    m = (jnp.dot(a0[:, :DH], wmp_ref[:DH, :], preferred_element_type=jnp.float32)
         + jnp.dot(a1[:, :DH], wmp_ref[DH:, :],
                   preferred_element_type=jnp.float32))
    h = jnp.maximum(m / deg + bmp_ref[...], 0.0)
    h = jnp.maximum(
        jnp.dot(h, w1_ref[...], preferred_element_type=jnp.float32)
        + b1_ref[...], 0.0)
    out_ref[...] = (
        jnp.dot(h, w2_ref[...], preferred_element_type=jnp.float32)
        + b2_ref[...])

  full = lambda shape: pl.BlockSpec(shape, lambda i: (0, 0))
  return pl.pallas_call(
      body,
      grid=(grid,),
      in_specs=[
          pl.BlockSpec((1, BLK, DS), lambda i: (0, i, 0)),
          pl.BlockSpec((1, BLK, DS), lambda i: (1, i, 0)),
          full((D_FEAT, D_FEAT)),
          full((1, D_FEAT)),
          full((D_FEAT, 256)),
          full((1, 256)),
          full((256, D_FEAT)),
          full((1, D_FEAT)),
      ],
      out_specs=pl.BlockSpec((BLK, D_FEAT), lambda i: (i, 0)),
      out_shape=jax.ShapeDtypeStruct((N_NODES, D_FEAT), jnp.float32),
  )(agg, agg, W_mp, b_mp, W1, b1, W2, b2)


@jax.jit
def kernel(features, edge_index, W_mp, b_mp, W1, b1, W2, b2):
  src = edge_index[0].astype(jnp.int32)
  dst = edge_index[1].astype(jnp.int32)
  src4 = jnp.concatenate([src, src + N_NODES]).reshape(NC * NS, NCHUNK, CHUNK)
  dst3 = dst.reshape(NS, NCHUNK, CHUNK)
  halves = features.reshape(N_NODES, 2, DH).transpose(1, 0, 2)
  x_bf = (halves.reshape(2 * N_NODES, DH // 32, 2, 16)
          .transpose(0, 1, 3, 2)
          .reshape(2 * N_NODES, DH)
          .astype(jnp.bfloat16))

  agg = _sc_aggregate(x_bf, src4, dst3)
  return _tc_head(agg, W_mp, b_mp.reshape(1, -1), W1, b1.reshape(1, -1),
                  W2, b2.reshape(1, -1))


# speedup vs baseline: 1.0047x; 1.0005x over previous
import functools

import jax
import jax.numpy as jnp
from jax import lax
from jax.experimental import pallas as pl
from jax.experimental.pallas import tpu as pltpu
from jax.experimental.pallas import tpu_sc as plsc

N_NODES = 10000
D_FEAT = 128
DH = 64
DS = 80
N_PAD = 10016
NC, NS = 2, 16
CHUNK = 125
NCHUNK = 160
assert NS * NCHUNK * CHUNK == 320000
NG = 4
ROWS_PER_TILE = N_PAD // NS


def _sc_aggregate(x_bf, src4, dst3):
  mesh = plsc.VectorSubcoreMesh(
      core_axis_name="c", subcore_axis_name="s", num_cores=NC, num_subcores=NS)

  @functools.partial(
      pl.kernel,
      out_type=jax.ShapeDtypeStruct((NC, N_PAD, DS), jnp.float32),
      mesh=mesh,
      compiler_params=pltpu.CompilerParams(
          use_tc_tiling_on_sc=False, needs_layout_passes=False),
      scratch_types=[
          pltpu.VMEM((NCHUNK, CHUNK), jnp.int32),
          pltpu.VMEM((NCHUNK, CHUNK), jnp.int32),
          [pltpu.VMEM((CHUNK, DH), jnp.bfloat16)] * NG,
          [pltpu.VMEM((CHUNK, DS), jnp.float32)] * 2,
          pltpu.VMEM_SHARED((N_PAD, DS), jnp.float32),
          [pltpu.SemaphoreType.DMA] * NG,
          [pltpu.SemaphoreType.DMA] * 2,
      ],
  )
  def k(x_hbm, src_hbm, dst_hbm, out_hbm,
        src_v, dst_v, gbuf, sbuf, acc, gsem, ssem):
    c = lax.axis_index("c")
    s = lax.axis_index("s")

    pltpu.sync_copy(src_hbm.at[c * NS + s], src_v)
    pltpu.sync_copy(dst_hbm.at[s], dst_v)
    for b in range(NG):
      pltpu.async_copy(x_hbm.at[src_v.at[b]], gbuf[b], gsem[b])

    def zero_row(r, _):
      for kk in range(DS // 16):
        sbuf[0][r, pl.ds(kk * 16, 16)] = jnp.zeros((16,), jnp.float32)
      return 0
    lax.fori_loop(0, CHUNK, zero_row, 0)
    base = s * ROWS_PER_TILE
    for i in range(ROWS_PER_TILE // CHUNK):
      pltpu.sync_copy(sbuf[0], acc.at[pl.ds(base + i * CHUNK, CHUNK)])
    rem = ROWS_PER_TILE % CHUNK
    if rem:
      pltpu.sync_copy(sbuf[0].at[pl.ds(0, rem)],
                      acc.at[pl.ds(base + ROWS_PER_TILE - rem, rem)])

    one0 = jnp.where(lax.iota(jnp.int32, 16) == 0, 1.0, 0.0).astype(jnp.float32)

    def ones_row(r, _):
      for sb in range(2):
        sbuf[sb][r, pl.ds(DH, 16)] = one0
      return 0
    lax.fori_loop(0, CHUNK, ones_row, 0)

    plsc.subcore_barrier()


    def unpack_chunk(b, sb):
      def rows5(r5, _):
        for dr in range(5):
          r = r5 * 5 + dr
          for g in range(DH // 32):
            lo, hi = plsc.unpack(gbuf[b][r, pl.ds(32 * g, 32)],
                                 format=plsc.PackFormat.INTERLEAVED)
            sbuf[sb][r, pl.ds(32 * g, 16)] = lo
            sbuf[sb][r, pl.ds(32 * g + 16, 16)] = hi
        return 0
      lax.fori_loop(0, CHUNK // 5, rows5, 0)

    def body(grp, _):
      for b in range(NG):
        j = grp * NG + b
        sb = b % 2
        pltpu.make_async_copy(x_hbm.at[src_v.at[j]], gbuf[b], gsem[b]).wait()

        @pl.when(j >= 2)
        def _():
          pltpu.make_async_copy(
              sbuf[sb], acc.at[dst_v.at[j - 2]], ssem[sb]).wait()

        unpack_chunk(b, sb)
        pltpu.async_copy(sbuf[sb], acc.at[dst_v.at[j]], ssem[sb], add=True)

        @pl.when(j + NG < NCHUNK)
        def _():
          pltpu.async_copy(x_hbm.at[src_v.at[j + NG]], gbuf[b], gsem[b])

      return 0

    lax.fori_loop(0, NCHUNK // NG, body, 0)
    for sb in range(2):
      pltpu.make_async_copy(
          sbuf[sb], acc.at[dst_v.at[NCHUNK - 2 + sb]], ssem[sb]).wait()

    plsc.subcore_barrier()
    pltpu.sync_copy(acc.at[pl.ds(base, ROWS_PER_TILE)],
                    out_hbm.at[c, pl.ds(base, ROWS_PER_TILE)])

  return k(x_bf, src4, dst3)


def _tc_head(agg, W_mp, b_mp, W1, b1, W2, b2):
  BLK = 2000
  grid = N_NODES // BLK

  def body(a0_ref, a1_ref, wmp_ref, bmp_ref, w1_ref, b1_ref, w2_ref, b2_ref, out_ref):
    a0 = a0_ref[0]
    a1 = a1_ref[0]
    deg = jnp.maximum(jnp.sum(a0[:, DH:], axis=1, keepdims=True), 1.0)
    m = (jnp.dot(a0[:, :DH], wmp_ref[:DH, :], preferred_element_type=jnp.float32)
         + jnp.dot(a1[:, :DH], wmp_ref[DH:, :],
                   preferred_element_type=jnp.float32))
    h = jnp.maximum(m / deg + bmp_ref[...], 0.0)
    h = jnp.maximum(
        jnp.dot(h, w1_ref[...], preferred_element_type=jnp.float32)
        + b1_ref[...], 0.0)
    out_ref[...] = (
        jnp.dot(h, w2_ref[...], preferred_element_type=jnp.float32)
        + b2_ref[...])

  full = lambda shape: pl.BlockSpec(shape, lambda i: (0, 0))
  return pl.pallas_call(
      body,
      grid=(grid,),
      in_specs=[
          pl.BlockSpec((1, BLK, DS), lambda i: (0, i, 0)),
          pl.BlockSpec((1, BLK, DS), lambda i: (1, i, 0)),
          full((D_FEAT, D_FEAT)),
          full((1, D_FEAT)),
          full((D_FEAT, 256)),
          full((1, 256)),
          full((256, D_FEAT)),
          full((1, D_FEAT)),
      ],
      out_specs=pl.BlockSpec((BLK, D_FEAT), lambda i: (i, 0)),
      out_shape=jax.ShapeDtypeStruct((N_NODES, D_FEAT), jnp.float32),
  )(agg, agg, W_mp, b_mp, W1, b1, W2, b2)


@jax.jit
def kernel(features, edge_index, W_mp, b_mp, W1, b1, W2, b2):
  src = edge_index[0].astype(jnp.int32)
  dst = edge_index[1].astype(jnp.int32)
  src4 = jnp.concatenate([src, src + N_NODES]).reshape(NC * NS, NCHUNK, CHUNK)
  dst3 = dst.reshape(NS, NCHUNK, CHUNK)
  halves = features.reshape(N_NODES, 2, DH).transpose(1, 0, 2)
  x_bf = (halves.reshape(2 * N_NODES, DH // 32, 2, 16)
          .transpose(0, 1, 3, 2)
          .reshape(2 * N_NODES, DH)
          .astype(jnp.bfloat16))

  agg = _sc_aggregate(x_bf, src4, dst3)
  return _tc_head(agg, W_mp, b_mp.reshape(1, -1), W1, b1.reshape(1, -1),
                  W2, b2.reshape(1, -1))
